# baseline (speedup 1.0000x reference)
import functools

import jax
import jax.numpy as jnp
from jax import lax
from jax.experimental import pallas as pl
from jax.experimental.pallas import tpu as pltpu
from jax.experimental.pallas import tpu_sc as plsc

BATCH = 4
SEQ = 8192
DM = 1024
CHUNK = 32
NBUF = 3
SC_BATCHES = 2
TC_BLOCK_S = 1024


@functools.cache
def _sc_copy_kernel():
    info = plsc.get_sparse_core_info()
    nw = info.num_cores * info.num_subcores
    rows_per = SEQ // nw
    nch = rows_per // CHUNK
    mesh = plsc.VectorSubcoreMesh(core_axis_name="c", subcore_axis_name="s")

    @functools.partial(
        pl.kernel,
        mesh=mesh,
        out_type=jax.ShapeDtypeStruct((BATCH, SEQ, DM), jnp.float32),
        scratch_types=(
            [pltpu.VMEM((CHUNK, DM), jnp.float32)] * NBUF
            + [pltpu.SemaphoreType.DMA] * (2 * NBUF)
        ),
    )
    def sc_copy(table_hbm, out_hbm, *scratch):
        bufs = scratch[:NBUF]
        lsem = scratch[NBUF : 2 * NBUF]
        ssem = scratch[2 * NBUF :]
        wid = lax.axis_index("s") * info.num_cores + lax.axis_index("c")
        base = wid * rows_per

        def row(k):
            return base + k * CHUNK

        loads = {}
        stores = {}
        for j in range(min(NBUF - 1, nch)):
            loads[j] = pltpu.async_copy(
                table_hbm.at[pl.ds(row(j), CHUNK)], bufs[j], lsem[j]
            )
        for k in range(nch):
            loads[k].wait()
            stores[k] = [
                pltpu.async_copy(
                    bufs[k % NBUF], out_hbm.at[b, pl.ds(row(k), CHUNK)], ssem[k % NBUF]
                )
                for b in range(SC_BATCHES)
            ]
            nxt = k + NBUF - 1
            if nxt < nch:
                if k >= 1:
                    for h in stores[k - 1]:
                        h.wait()
                loads[nxt] = pltpu.async_copy(
                    table_hbm.at[pl.ds(row(nxt), CHUNK)], bufs[nxt % NBUF], lsem[nxt % NBUF]
                )
        for k in range(max(0, nch - NBUF), nch):
            for h in stores[k]:
                h.wait()

    return sc_copy


def _tc_body(t_ref, prev_ref, o_ref):
    del prev_ref
    o_ref[...] = jnp.broadcast_to(t_ref[...][None], o_ref.shape)


def _tc_fill(table, prev):
    tc_batches = BATCH - SC_BATCHES
    grid = (SEQ // TC_BLOCK_S,)
    return pl.pallas_call(
        _tc_body,
        grid=grid,
        in_specs=[
            pl.BlockSpec((TC_BLOCK_S, DM), lambda i: (i, 0)),
            pl.BlockSpec(memory_space=pl.ANY),
        ],
        out_specs=pl.BlockSpec(
            (tc_batches, TC_BLOCK_S, DM), lambda i: (SC_BATCHES // tc_batches, i, 0)
        ),
        out_shape=jax.ShapeDtypeStruct((BATCH, SEQ, DM), jnp.float32),
        input_output_aliases={1: 0},
    )(table, prev)


def kernel(x, table):
    del x
    partial = _sc_copy_kernel()(table)
    return _tc_fill(table, partial)


# speedup vs baseline: 1.0990x; 1.0990x over previous
import functools

import jax
import jax.numpy as jnp
from jax import lax
from jax.experimental import pallas as pl
from jax.experimental.pallas import tpu as pltpu
from jax.experimental.pallas import tpu_sc as plsc

BATCH = 4
SEQ = 8192
DM = 1024
CHUNK = 16


NBUF = 6


@functools.cache
def _sc_copy_kernel():
    info = plsc.get_sparse_core_info()
    nw = info.num_cores * info.num_subcores
    rows_per = SEQ // nw
    nch = rows_per // CHUNK
    mesh = plsc.VectorSubcoreMesh(core_axis_name="c", subcore_axis_name="s")

    @functools.partial(
        pl.kernel,
        mesh=mesh,
        out_type=jax.ShapeDtypeStruct((BATCH, SEQ, DM), jnp.float32),
        scratch_types=(
            [pltpu.VMEM((CHUNK, DM), jnp.float32)] * NBUF
            + [pltpu.SemaphoreType.DMA] * (2 * NBUF)
        ),
    )
    def sc_copy(table_hbm, out_hbm, *scratch):
        bufs = scratch[:NBUF]
        lsem = scratch[NBUF : 2 * NBUF]
        ssem = scratch[2 * NBUF :]
        wid = lax.axis_index("s") * info.num_cores + lax.axis_index("c")
        base = wid * rows_per

        def row(k):
            return base + k * CHUNK

        loads = {}
        stores = {}
        for j in range(min(NBUF - 1, nch)):
            loads[j] = pltpu.async_copy(
                table_hbm.at[pl.ds(row(j), CHUNK)], bufs[j], lsem[j]
            )
        for k in range(nch):
            loads[k].wait()
            stores[k] = [
                pltpu.async_copy(
                    bufs[k % NBUF], out_hbm.at[b, pl.ds(row(k), CHUNK)], ssem[k % NBUF]
                )
                for b in range(BATCH)
            ]
            nxt = k + NBUF - 1
            if nxt < nch:
                if k >= 1:
                    for h in stores[k - 1]:
                        h.wait()
                loads[nxt] = pltpu.async_copy(
                    table_hbm.at[pl.ds(row(nxt), CHUNK)], bufs[nxt % NBUF], lsem[nxt % NBUF]
                )
        for k in range(max(0, nch - NBUF), nch):
            for h in stores[k]:
                h.wait()

    return sc_copy


def kernel(x, table):
    del x
    return _sc_copy_kernel()(table)


# speedup vs baseline: 1.1265x; 1.0250x over previous
import functools

import jax
import jax.numpy as jnp
from jax import lax
from jax.experimental import pallas as pl
from jax.experimental.pallas import tpu as pltpu
from jax.experimental.pallas import tpu_sc as plsc

BATCH = 4
SEQ = 8192
DM = 1024
CHUNK = 48


NBUF = 2


@functools.cache
def _sc_copy_kernel():
    info = plsc.get_sparse_core_info()
    nw = info.num_cores * info.num_subcores
    rows_per = SEQ // nw
    sizes = [CHUNK] * (rows_per // CHUNK)
    if rows_per % CHUNK:
        sizes.append(rows_per % CHUNK)
    offs = [sum(sizes[:i]) for i in range(len(sizes))]
    nch = len(sizes)
    mesh = plsc.VectorSubcoreMesh(core_axis_name="c", subcore_axis_name="s")

    @functools.partial(
        pl.kernel,
        mesh=mesh,
        out_type=jax.ShapeDtypeStruct((BATCH, SEQ, DM), jnp.float32),
        scratch_types=(
            [pltpu.VMEM((CHUNK, DM), jnp.float32)] * NBUF
            + [pltpu.SemaphoreType.DMA] * (2 * NBUF)
        ),
    )
    def sc_copy(table_hbm, out_hbm, *scratch):
        bufs = scratch[:NBUF]
        lsem = scratch[NBUF : 2 * NBUF]
        ssem = scratch[2 * NBUF :]
        wid = lax.axis_index("s") * info.num_cores + lax.axis_index("c")
        base = wid * rows_per

        def row(k):
            return base + offs[k]

        def buf(k):
            b = bufs[k % NBUF]
            return b if sizes[k] == CHUNK else b.at[pl.ds(0, sizes[k])]

        loads = {}
        stores = {}
        for j in range(min(NBUF - 1, nch)):
            loads[j] = pltpu.async_copy(
                table_hbm.at[pl.ds(row(j), sizes[j])], buf(j), lsem[j]
            )
        for k in range(nch):
            loads[k].wait()
            stores[k] = [
                pltpu.async_copy(
                    buf(k), out_hbm.at[b, pl.ds(row(k), sizes[k])], ssem[k % NBUF]
                )
                for b in range(BATCH)
            ]
            nxt = k + NBUF - 1
            if nxt < nch:
                if k >= 1:
                    for h in stores[k - 1]:
                        h.wait()
                loads[nxt] = pltpu.async_copy(
                    table_hbm.at[pl.ds(row(nxt), sizes[nxt])], buf(nxt), lsem[nxt % NBUF]
                )
        for k in range(max(0, nch - NBUF), nch):
            for h in stores[k]:
                h.wait()

    return sc_copy


def kernel(x, table):
    del x
    return _sc_copy_kernel()(table)
